# baseline (speedup 1.0000x reference)
import functools

import jax
import jax.numpy as jnp
from jax import lax
from jax.experimental import pallas as pl
from jax.experimental.pallas import tpu as pltpu
from jax.experimental.pallas import tpu_sc as plsc

_TABLE_SIZES = (100003, 100019, 100043, 100057, 100069, 100103, 100109,
                100129)
_VMAX = max(_TABLE_SIZES)
_NUM_HEADS = 8
_EMBED_DIM = 32
_NC = 2
_NS = 16
_NW = _NC * _NS
_CHUNK = 128


@functools.lru_cache(maxsize=None)
def _build(B: int, S: int):
    assert B == _NW * _CHUNK
    assert S % 2 == 0
    mesh = plsc.VectorSubcoreMesh(core_axis_name="c", subcore_axis_name="s")

    @functools.partial(
        pl.kernel,
        out_type=jax.ShapeDtypeStruct((S, B, _NUM_HEADS * _EMBED_DIM),
                                      jnp.float32),
        mesh=mesh,
        compiler_params=pltpu.CompilerParams(use_tc_tiling_on_sc=False),
        scratch_types=[
            pltpu.VMEM((2, _NUM_HEADS, _CHUNK), jnp.int32),
            pltpu.VMEM((2, _NUM_HEADS, _CHUNK, _EMBED_DIM), jnp.float32),
            pltpu.SemaphoreType.DMA((2,)),
            pltpu.SemaphoreType.DMA((2,)),
        ],
    )
    def gather_kernel(idx_hbm, tab_hbm, out_hbm, idx_v, rows_v, gsem, osem):
        wid = lax.axis_index("s") * _NC + lax.axis_index("c")
        b0 = wid * _CHUNK

        def fire(s, b):
            pltpu.sync_copy(idx_hbm.at[s, :, wid], idx_v.at[b])
            for h in range(_NUM_HEADS):
                pltpu.async_copy(tab_hbm.at[h].at[idx_v.at[b, h]],
                                 rows_v.at[b, h], gsem.at[b])

        def drain_gather(b):
            for h in range(_NUM_HEADS):
                pltpu.make_async_copy(tab_hbm.at[h].at[idx_v.at[b, h]],
                                      rows_v.at[b, h], gsem.at[b]).wait()

        def write_out(s, b):
            for h in range(_NUM_HEADS):
                pltpu.async_copy(
                    rows_v.at[b, h],
                    out_hbm.at[s, pl.ds(b0, _CHUNK),
                               pl.ds(h * _EMBED_DIM, _EMBED_DIM)],
                    osem.at[b])

        def wait_out(s, b):
            for h in range(_NUM_HEADS):
                pltpu.make_async_copy(
                    rows_v.at[b, h],
                    out_hbm.at[s, pl.ds(b0, _CHUNK),
                               pl.ds(h * _EMBED_DIM, _EMBED_DIM)],
                    osem.at[b]).wait()

        fire(0, 0)

        def body(j2, carry):
            for b in range(2):
                s = j2 * 2 + b
                nb = 1 - b

                @pl.when(s + 1 < S)
                def _():
                    @pl.when(s >= 1)
                    def _():
                        wait_out(s - 1, nb)
                    fire(s + 1, nb)

                drain_gather(b)
                write_out(s, b)
            return carry

        lax.fori_loop(0, S // 2, body, 0)
        wait_out(S - 2, 0)
        wait_out(S - 1, 1)

    return gather_kernel


def kernel(hash_indices, table_0, table_1, table_2, table_3, table_4,
           table_5, table_6, table_7):
    B, S, H = hash_indices.shape
    tables = [table_0, table_1, table_2, table_3, table_4, table_5, table_6,
              table_7]
    tab = jnp.stack([
        jnp.pad(t, ((0, _VMAX - t.shape[0]), (0, 0))) for t in tables
    ])
    idx4 = jnp.transpose(hash_indices, (1, 2, 0)).reshape(
        S, H, B // _CHUNK, _CHUNK)
    out = _build(B, S)(idx4, tab)
    return jnp.transpose(out, (1, 0, 2))


# speedup vs baseline: 2.7664x; 2.7664x over previous
import functools

import jax
import jax.numpy as jnp
from jax import lax
from jax.experimental import pallas as pl
from jax.experimental.pallas import tpu as pltpu
from jax.experimental.pallas import tpu_sc as plsc

_TABLE_SIZES = (100003, 100019, 100043, 100057, 100069, 100103, 100109,
                100129)
_NUM_HEADS = 8
_EMBED_DIM = 32
_NC = 2
_NS = 16
_NW = _NC * _NS
_CHUNK = 128


@functools.lru_cache(maxsize=None)
def _build(B: int, S: int):
    assert B == _NW * _CHUNK
    assert S % 2 == 0
    mesh = plsc.VectorSubcoreMesh(core_axis_name="c", subcore_axis_name="s")

    @functools.partial(
        pl.kernel,
        out_type=jax.ShapeDtypeStruct((S, B, _NUM_HEADS * _EMBED_DIM),
                                      jnp.float32),
        mesh=mesh,
        compiler_params=pltpu.CompilerParams(use_tc_tiling_on_sc=False),
        scratch_types=[
            pltpu.VMEM((2, _NUM_HEADS, _CHUNK), jnp.int32),
            pltpu.VMEM((2, _NUM_HEADS, _CHUNK, _EMBED_DIM), jnp.float32),
            pltpu.SemaphoreType.DMA((2,)),
            pltpu.SemaphoreType.DMA((2,)),
        ],
    )
    def gather_kernel(idx_hbm, t0, t1, t2, t3, t4, t5, t6, t7, out_hbm,
                      idx_v, rows_v, gsem, osem):
        tabs = [t0, t1, t2, t3, t4, t5, t6, t7]
        wid = lax.axis_index("s") * _NC + lax.axis_index("c")
        b0 = wid * _CHUNK

        def fire(s, b):
            pltpu.sync_copy(idx_hbm.at[s, wid], idx_v.at[b])
            for h in range(_NUM_HEADS):
                pltpu.async_copy(tabs[h].at[idx_v.at[b, h]],
                                 rows_v.at[b, h], gsem.at[b])

        def drain_gather(b):
            for h in range(_NUM_HEADS):
                pltpu.make_async_copy(tabs[h].at[idx_v.at[b, h]],
                                      rows_v.at[b, h], gsem.at[b]).wait()

        def write_out(s, b):
            for h in range(_NUM_HEADS):
                pltpu.async_copy(
                    rows_v.at[b, h],
                    out_hbm.at[s, pl.ds(b0, _CHUNK),
                               pl.ds(h * _EMBED_DIM, _EMBED_DIM)],
                    osem.at[b])

        def wait_out(s, b):
            for h in range(_NUM_HEADS):
                pltpu.make_async_copy(
                    rows_v.at[b, h],
                    out_hbm.at[s, pl.ds(b0, _CHUNK),
                               pl.ds(h * _EMBED_DIM, _EMBED_DIM)],
                    osem.at[b]).wait()

        fire(0, 0)

        def body(j2, carry):
            for b in range(2):
                s = j2 * 2 + b
                nb = 1 - b

                @pl.when(s + 1 < S)
                def _():
                    @pl.when(s >= 1)
                    def _():
                        wait_out(s - 1, nb)
                    fire(s + 1, nb)

                drain_gather(b)
                write_out(s, b)
            return carry

        lax.fori_loop(0, S // 2, body, 0)
        wait_out(S - 2, 0)
        wait_out(S - 1, 1)

    return gather_kernel


def kernel(hash_indices, table_0, table_1, table_2, table_3, table_4,
           table_5, table_6, table_7):
    B, S, H = hash_indices.shape
    idx4 = jnp.transpose(hash_indices, (1, 0, 2)).reshape(
        S, B // _CHUNK, _CHUNK, H).transpose(0, 1, 3, 2)
    out = _build(B, S)(idx4, table_0, table_1, table_2, table_3, table_4,
                       table_5, table_6, table_7)
    return jnp.transpose(out, (1, 0, 2))
